# baseline (speedup 1.0000x reference)
import functools
import jax
import jax.numpy as jnp
from jax import lax
from jax.experimental import pallas as pl
from jax.experimental.pallas import tpu as pltpu
from jax.experimental.pallas import tpu_sc as plsc

D = 64
SCALE = 8.0
NC, NS = 2, 16
NW = NC * NS
SEQ = 200
BATCH = 4096
B = BATCH * SEQ
BPW = B // NW
IDX_MINOR = 128
NIDXROW = BPW // IDX_MINOR
CHUNK = 256
GPC = CHUNK // IDX_MINOR
NCH = B // CHUNK
NCHW = NCH // NW
CPS = BATCH // CHUNK
TD, DR = D // 8, 8
TBL = CHUNK // 128

_mesh = plsc.VectorSubcoreMesh(
    core_axis_name="c", subcore_axis_name="s", num_cores=NC, num_subcores=NS
)


@functools.partial(
    pl.kernel,
    out_type=jax.ShapeDtypeStruct((SEQ, TD, BATCH // 128, DR, 128), jnp.float32),
    mesh=_mesh,
    scratch_types=[
        pltpu.VMEM((NIDXROW, IDX_MINOR), jnp.int32),
        [pltpu.VMEM((CHUNK, D), jnp.float32) for _ in range(2)],
        [pltpu.VMEM((TD, TBL, DR, 128), jnp.float32) for _ in range(2)],
        [pltpu.SemaphoreType.DMA for _ in range(2)],
        [pltpu.SemaphoreType.DMA for _ in range(2)],
    ],
    compiler_params=pltpu.CompilerParams(
        use_tc_tiling_on_sc=False, needs_layout_passes=False
    ),
)
def _emb_lookup(x_hbm, lut_hbm, out_hbm, idx_v, rows, stage, gsem, osem):
    wid = lax.axis_index("s") * NC + lax.axis_index("c")

    pltpu.sync_copy(x_hbm.at[wid], idx_v)

    def fire_gathers(k, r):
        for j in range(GPC):
            pltpu.async_copy(
                lut_hbm.at[idx_v.at[k * GPC + j]],
                rows[r].at[pl.ds(j * IDX_MINOR, IDX_MINOR)],
                gsem[r],
            )

    def drain_gathers(k, r):
        for j in range(GPC):
            pltpu.make_async_copy(
                lut_hbm.at[idx_v.at[k * GPC + j]],
                rows[r].at[pl.ds(j * IDX_MINOR, IDX_MINOR)],
                gsem[r],
            ).wait()

    def transform(r):
        iota = lax.iota(jnp.int32, 16)

        @pl.loop(0, D)
        def _outer(o):
            td = o >> 3
            dr = o & 7
            col = jnp.full((16,), o, jnp.int32)

            @pl.loop(0, CHUNK // 16, init_carry=iota, unroll=8)
            def _inner(i, row_ids):
                v = plsc.load_gather(rows[r], [row_ids, col])
                stage[r][td, i >> 3, dr, pl.ds((i & 7) * 16, 16)] = v * SCALE
                return row_ids + 16

    def _dst(c, td):
        s = c >> 4
        tb0 = (c & 15) * TBL
        return out_hbm.at[s, td, pl.ds(tb0, TBL)]

    def fire_write(c, r):
        for td in range(TD):
            pltpu.async_copy(stage[r].at[td], _dst(c, td), osem[r])

    def drain_write(c, r):
        for td in range(TD):
            pltpu.make_async_copy(stage[r].at[td], _dst(c, td), osem[r]).wait()

    c0 = wid * NCHW
    fire_gathers(0, 0)
    fire_gathers(1, 1)

    @pl.loop(0, NCHW // 2)
    def _step(j):
        for r in range(2):
            k = 2 * j + r
            drain_gathers(k, r)

            @pl.when(k >= 2)
            def _():
                drain_write(c0 + k - 2, r)

            transform(r)

            @pl.when(k + 2 < NCHW)
            def _():
                fire_gathers(k + 2, r)

            fire_write(c0 + k, r)

    drain_write(c0 + NCHW - 2, 0)
    drain_write(c0 + NCHW - 1, 1)


def kernel(x, lut):
    xf = x.T.reshape(NW, NIDXROW, IDX_MINOR).astype(jnp.int32)
    out5 = _emb_lookup(xf, lut)
    return out5.transpose(2, 4, 0, 1, 3).reshape(BATCH, SEQ, D)
